# baseline (speedup 1.0000x reference)
import functools

import jax
import jax.numpy as jnp
from jax import lax
from jax.experimental import pallas as pl
from jax.experimental.pallas import tpu as pltpu
from jax.experimental.pallas import tpu_sc as plsc

D_MODEL = 64
_NC = 2
_NS = 16
_NW = _NC * _NS
_SUB = 128
_K = 8
_CHUNK = _K * _SUB
_L = 16


@functools.partial(jax.jit, static_argnames=("n_iter",))
def _gather(table1d, idx2d, n_iter):
    b = idx2d.shape[0] * idx2d.shape[1]
    b_per_w = b // _NW
    mesh = plsc.VectorSubcoreMesh(core_axis_name="c", subcore_axis_name="s")

    @functools.partial(
        pl.kernel,
        mesh=mesh,
        compiler_params=pltpu.CompilerParams(use_tc_tiling_on_sc=False),
        out_type=jax.ShapeDtypeStruct((b, D_MODEL), jnp.float32),
        scratch_types=[
            pltpu.VMEM((_K, _SUB), jnp.int32),
            pltpu.VMEM((_CHUNK,), jnp.int32),
            pltpu.VMEM((_CHUNK, D_MODEL), jnp.float32),
            pltpu.SemaphoreType.DMA,
        ],
    )
    def gather_kernel(table_hbm, idx_hbm, out_hbm, idx_v, off_v, rows_v, sem):
        wid = lax.axis_index("s") * _NC + lax.axis_index("c")
        base = wid * b_per_w

        def body(i, carry):
            off = pl.multiple_of(base + i * _CHUNK, _CHUNK)
            row0 = pl.multiple_of(off // _SUB, _K)
            pltpu.sync_copy(idx_hbm.at[pl.ds(row0, _K)], idx_v)
            copies = [
                pltpu.async_copy(
                    table_hbm.at[idx_v.at[j]],
                    rows_v.at[pl.ds(j * _SUB, _SUB)],
                    sem,
                )
                for j in range(_K)
            ]
            for c in copies:
                c.wait()
            pltpu.sync_copy(rows_v, out_hbm.at[pl.ds(off, _CHUNK)])
            return carry

        lax.fori_loop(0, n_iter, body, 0)

    return gather_kernel(table1d, idx2d)


def kernel(x, table):
    s0, s1 = x.shape
    b = s0 * s1
    idx2d = x.reshape(b // _SUB, _SUB).astype(jnp.int32)
    n_iter = b // (_NW * _CHUNK)
    out = _gather(table, idx2d, n_iter)
    return out.reshape(s0, s1, D_MODEL)
